# baseline (speedup 1.0000x reference)
import functools
import jax
import jax.numpy as jnp
from jax.experimental import pallas as pl
from jax.experimental.pallas import tpu as pltpu

NBLOCKS = 4
BM = 512
BK = 512


def _dot(a, b):
    return jax.lax.dot_general(a, b, (((1,), (0,)), ((), ())),
                               preferred_element_type=jnp.float32)


def _apply_body(a_ref, c_ref, z_ref, o_ref, *, kb):
    k = pl.program_id(1)

    @pl.when(k == 0)
    def _():
        o_ref[...] = z_ref[...]

    o_ref[...] += _dot(a_ref[...], c_ref[...])


def _apply_tanh_z_body(a_ref, c_ref, z_ref, b_ref, w_ref, o_ref, acc_ref, *, kb):
    k = pl.program_id(1)

    @pl.when(k == 0)
    def _():
        acc_ref[...] = z_ref[...]

    acc_ref[...] += _dot(a_ref[...], c_ref[...])

    @pl.when(k == kb - 1)
    def _():
        h = jnp.tanh(acc_ref[...] + b_ref[...])
        o_ref[...] = _dot(h, w_ref[...])


def _apply_final_body(a_ref, c_ref, z_ref, b_ref, w_ref, bo_ref, o_ref, acc_ref, *, kb):
    k = pl.program_id(1)

    @pl.when(k == 0)
    def _():
        acc_ref[...] = z_ref[...]

    acc_ref[...] += _dot(a_ref[...], c_ref[...])

    @pl.when(k == kb - 1)
    def _():
        h = jnp.tanh(acc_ref[...] + b_ref[...])
        o = _dot(h, w_ref[...]) + bo_ref[...]
        nrm = jnp.sqrt(jnp.sum(o * o, axis=1, keepdims=True))
        o_ref[...] = o / jnp.maximum(nrm, 1e-12)


_PARAMS = pltpu.CompilerParams(dimension_semantics=("parallel", "arbitrary"))


def _apply(A, cur, Z):
    n, nh = cur.shape
    mb, kb = n // BM, n // BK
    grid = (mb, kb)
    return pl.pallas_call(
        functools.partial(_apply_body, kb=kb),
        grid=grid,
        in_specs=[
            pl.BlockSpec((BM, BK), lambda i, k: (i, k)),
            pl.BlockSpec((BK, nh), lambda i, k: (k, 0)),
            pl.BlockSpec((BM, nh), lambda i, k: (i, 0)),
        ],
        out_specs=pl.BlockSpec((BM, nh), lambda i, k: (i, 0)),
        out_shape=jax.ShapeDtypeStruct((n, nh), jnp.float32),
        compiler_params=_PARAMS,
    )(A, cur, Z)


def _apply_tanh_z(A, cur, Z, b, Wcat):
    n, nh = cur.shape
    nw = Wcat.shape[1]
    mb, kb = n // BM, n // BK
    return pl.pallas_call(
        functools.partial(_apply_tanh_z_body, kb=kb),
        grid=(mb, kb),
        in_specs=[
            pl.BlockSpec((BM, BK), lambda i, k: (i, k)),
            pl.BlockSpec((BK, nh), lambda i, k: (k, 0)),
            pl.BlockSpec((BM, nh), lambda i, k: (i, 0)),
            pl.BlockSpec((1, nh), lambda i, k: (0, 0)),
            pl.BlockSpec((nh, nw), lambda i, k: (0, 0)),
        ],
        out_specs=pl.BlockSpec((BM, nw), lambda i, k: (i, 0)),
        out_shape=jax.ShapeDtypeStruct((n, nw), jnp.float32),
        scratch_shapes=[pltpu.VMEM((BM, nh), jnp.float32)],
        compiler_params=_PARAMS,
    )(A, cur, Z, b, Wcat)


def _apply_final(A, cur, Z, b, Wout, bout):
    n, nh = cur.shape
    nc = Wout.shape[1]
    mb, kb = n // BM, n // BK
    return pl.pallas_call(
        functools.partial(_apply_final_body, kb=kb),
        grid=(mb, kb),
        in_specs=[
            pl.BlockSpec((BM, BK), lambda i, k: (i, k)),
            pl.BlockSpec((BK, nh), lambda i, k: (k, 0)),
            pl.BlockSpec((BM, nh), lambda i, k: (i, 0)),
            pl.BlockSpec((1, nh), lambda i, k: (0, 0)),
            pl.BlockSpec((nh, nc), lambda i, k: (0, 0)),
            pl.BlockSpec((1, nc), lambda i, k: (0, 0)),
        ],
        out_specs=pl.BlockSpec((BM, nc), lambda i, k: (i, 0)),
        out_shape=jax.ShapeDtypeStruct((n, nc), jnp.float32),
        scratch_shapes=[pltpu.VMEM((BM, nh), jnp.float32)],
        compiler_params=_PARAMS,
    )(A, cur, Z, b, Wout, bout)


def _zmat_body(x_ref, w_ref, o_ref):
    o_ref[...] = _dot(x_ref[...], w_ref[...])


def _zmat(X, Wcat):
    n, f = X.shape
    nw = Wcat.shape[1]
    mb = n // BM
    return pl.pallas_call(
        _zmat_body,
        grid=(mb,),
        in_specs=[
            pl.BlockSpec((BM, f), lambda i: (i, 0)),
            pl.BlockSpec((f, nw), lambda i: (0, 0)),
        ],
        out_specs=pl.BlockSpec((BM, nw), lambda i: (i, 0)),
        out_shape=jax.ShapeDtypeStruct((n, nw), jnp.float32),
        compiler_params=pltpu.CompilerParams(dimension_semantics=("parallel",)),
    )(X, Wcat)


def kernel(x, adj, features, W0, b0, W1, b1, W2, b2, Wout, bout):
    n = adj.shape[0]
    nh = Wout.shape[0]
    nfeat = features.shape[1]

    W0cat = jnp.concatenate(
        [W0[k * nfeat:(k + 1) * nfeat, :] for k in range(NBLOCKS)], axis=1)
    W1cat = jnp.concatenate(
        [W1[k * nh:(k + 1) * nh, :] for k in range(NBLOCKS)], axis=1)
    W2cat = jnp.concatenate(
        [W2[k * nh:(k + 1) * nh, :] for k in range(NBLOCKS)], axis=1)

    Z = _zmat(features, W0cat)
    for Wcat_next, b in ((W1cat, b0), (W2cat, b1)):
        t = _apply(adj, Z[:, 3 * nh:], Z[:, 2 * nh:3 * nh])
        t = _apply(adj, t, Z[:, nh:2 * nh])
        Z = _apply_tanh_z(adj, t, Z[:, :nh], b.reshape(1, -1), Wcat_next)
    t = _apply(adj, Z[:, 3 * nh:], Z[:, 2 * nh:3 * nh])
    t = _apply(adj, t, Z[:, nh:2 * nh])
    return _apply_final(adj, t, Z[:, :nh], b2.reshape(1, -1), Wout,
                        bout.reshape(1, -1))


# speedup vs baseline: 2.7671x; 2.7671x over previous
import jax
import jax.numpy as jnp
from jax.experimental import pallas as pl

NBLOCKS = 4


def _dot(a, b):
    return jax.lax.dot_general(a, b, (((1,), (0,)), ((), ())),
                               preferred_element_type=jnp.float32)


def _krylov_body(adj_ref, feat_ref, w0_ref, b0_ref, w1_ref, b1_ref,
                 w2_ref, b2_ref, wout_ref, bout_ref, out_ref):
    A = adj_ref[...]
    nfeat = feat_ref.shape[1]
    nhid = w0_ref.shape[1]

    cur = feat_ref[...]
    acc = _dot(cur, w0_ref[0:nfeat, :])
    for k in range(1, NBLOCKS):
        cur = _dot(A, cur).astype(jnp.bfloat16)
        acc = acc + _dot(cur, w0_ref[k * nfeat:(k + 1) * nfeat, :])
    h = jnp.tanh(acc + b0_ref[...]).astype(jnp.bfloat16)

    for w_ref, b_ref in ((w1_ref, b1_ref), (w2_ref, b2_ref)):
        cur = h
        acc = _dot(cur, w_ref[0:nhid, :])
        for k in range(1, NBLOCKS):
            cur = _dot(A, cur).astype(jnp.bfloat16)
            acc = acc + _dot(cur, w_ref[k * nhid:(k + 1) * nhid, :])
        h = jnp.tanh(acc + b_ref[...]).astype(jnp.bfloat16)

    o = _dot(h, wout_ref[...]) + bout_ref[...]
    nrm = jnp.sqrt(jnp.sum(o * o, axis=1, keepdims=True))
    out_ref[...] = o / jnp.maximum(nrm, 1e-12)


def kernel(x, adj, features, W0, b0, W1, b1, W2, b2, Wout, bout):
    n = adj.shape[0]
    nclass = Wout.shape[1]
    return pl.pallas_call(
        _krylov_body,
        out_shape=jax.ShapeDtypeStruct((n, nclass), jnp.float32),
    )(adj.astype(jnp.bfloat16), features.astype(jnp.bfloat16),
      W0.astype(jnp.bfloat16), b0.reshape(1, -1),
      W1.astype(jnp.bfloat16), b1.reshape(1, -1),
      W2.astype(jnp.bfloat16), b2.reshape(1, -1),
      Wout.astype(jnp.bfloat16), bout.reshape(1, -1))


# speedup vs baseline: 3.3872x; 1.2241x over previous
import jax
import jax.numpy as jnp
from jax.experimental import pallas as pl
from jax.experimental.pallas import tpu as pltpu

NBLOCKS = 4
NCHUNKS = 4


def _dot(a, b):
    return jax.lax.dot_general(a, b, (((1,), (0,)), ((), ())),
                               preferred_element_type=jnp.float32)


def _krylov_body(adj_hbm, feat_ref, w0_ref, b0_ref, w1_ref, b1_ref,
                 w2_ref, b2_ref, wout_ref, bout_ref, out_ref, a_vmem, sems):
    n = adj_hbm.shape[0]
    nfeat = feat_ref.shape[1]
    nhid = w0_ref.shape[1]
    rows = n // NCHUNKS

    copies = [
        pltpu.make_async_copy(
            adj_hbm.at[pl.ds(c * rows, rows), :],
            a_vmem.at[pl.ds(c * rows, rows), :],
            sems.at[c])
        for c in range(NCHUNKS)
    ]
    for cp in copies:
        cp.start()

    x = feat_ref[...]
    zs = [_dot(x, w0_ref[k * nfeat:(k + 1) * nfeat, :]) for k in range(NBLOCKS)]

    parts = []
    for c in range(NCHUNKS):
        copies[c].wait()
        parts.append(_dot(a_vmem[pl.ds(c * rows, rows), :], zs[3]))
    acc = zs[2] + jnp.concatenate(parts, axis=0)

    A = a_vmem[...]
    acc = zs[1] + _dot(A, acc)
    acc = zs[0] + _dot(A, acc)
    h = jnp.tanh(acc + b0_ref[...])

    for w_ref, b_ref in ((w1_ref, b1_ref), (w2_ref, b2_ref)):
        cur = h
        acc = _dot(cur, w_ref[0:nhid, :])
        for k in range(1, NBLOCKS):
            cur = _dot(A, cur)
            acc = acc + _dot(cur, w_ref[k * nhid:(k + 1) * nhid, :])
        h = jnp.tanh(acc + b_ref[...])

    o = _dot(h, wout_ref[...]) + bout_ref[...]
    nrm = jnp.sqrt(jnp.sum(o * o, axis=1, keepdims=True))
    out_ref[...] = o / jnp.maximum(nrm, 1e-12)


def kernel(x, adj, features, W0, b0, W1, b1, W2, b2, Wout, bout):
    n = adj.shape[0]
    nclass = Wout.shape[1]
    vmem = pl.BlockSpec(memory_space=pltpu.VMEM)
    return pl.pallas_call(
        _krylov_body,
        in_specs=[pl.BlockSpec(memory_space=pl.ANY)] + [vmem] * 9,
        out_specs=vmem,
        out_shape=jax.ShapeDtypeStruct((n, nclass), jnp.float32),
        scratch_shapes=[pltpu.VMEM((n, n), jnp.float32),
                        pltpu.SemaphoreType.DMA((NCHUNKS,))],
    )(adj, features, W0, b0.reshape(1, -1), W1, b1.reshape(1, -1),
      W2, b2.reshape(1, -1), Wout, bout.reshape(1, -1))


# speedup vs baseline: 3.5373x; 1.0443x over previous
import jax
import jax.numpy as jnp
from jax.experimental import pallas as pl

NBLOCKS = 4


def _dot(a, b):
    return jax.lax.dot_general(a, b, (((1,), (0,)), ((), ())),
                               preferred_element_type=jnp.float32)


def _krylov_body(adj_ref, feat_ref, w0_ref, b0_ref, w1_ref, b1_ref,
                 w2_ref, b2_ref, wout_ref, bout_ref, out_ref):
    A = adj_ref[...]
    nfeat = feat_ref.shape[1]
    nhid = w0_ref.shape[1]

    curs = [feat_ref[...]]
    for k in range(1, NBLOCKS):
        curs.append(_dot(A, curs[-1]))
    feat = jnp.concatenate(curs, axis=1)
    h = jnp.tanh(_dot(feat, w0_ref[...]) + b0_ref[...])

    for w_ref, b_ref in ((w1_ref, b1_ref), (w2_ref, b2_ref)):
        curs = [h]
        for k in range(1, NBLOCKS):
            curs.append(_dot(A, curs[-1]))
        feat = jnp.concatenate(curs, axis=1)
        h = jnp.tanh(_dot(feat, w_ref[...]) + b_ref[...])

    o = _dot(h, wout_ref[...]) + bout_ref[...]
    nrm = jnp.sqrt(jnp.sum(o * o, axis=1, keepdims=True))
    out_ref[...] = o / jnp.maximum(nrm, 1e-12)


def kernel(x, adj, features, W0, b0, W1, b1, W2, b2, Wout, bout):
    n = adj.shape[0]
    nclass = Wout.shape[1]
    return pl.pallas_call(
        _krylov_body,
        out_shape=jax.ShapeDtypeStruct((n, nclass), jnp.float32),
    )(adj, features, W0, b0.reshape(1, -1), W1, b1.reshape(1, -1),
      W2, b2.reshape(1, -1), Wout, bout.reshape(1, -1))


# speedup vs baseline: 3.5452x; 1.0022x over previous
import jax
import jax.numpy as jnp
from jax.experimental import pallas as pl

NBLOCKS = 4


def _dot(a, b):
    return jax.lax.dot_general(a, b, (((1,), (0,)), ((), ())),
                               preferred_element_type=jnp.float32)


def _krylov_body(adj_ref, feat_ref, w0_ref, b0_ref, w1_ref, b1_ref,
                 w2_ref, b2_ref, wout_ref, bout_ref, out_ref):
    A = adj_ref[...]
    nfeat = feat_ref.shape[1]
    nhid = w0_ref.shape[1]

    curs = [feat_ref[...]]
    for k in range(1, NBLOCKS):
        curs.append(_dot(A, curs[-1]))
    acc = b0_ref[...]
    for k in range(NBLOCKS):
        acc = acc + _dot(curs[k], w0_ref[k * nfeat:(k + 1) * nfeat, :])
    h = jnp.tanh(acc)

    for w_ref, b_ref in ((w1_ref, b1_ref), (w2_ref, b2_ref)):
        curs = [h]
        for k in range(1, NBLOCKS):
            curs.append(_dot(A, curs[-1]))
        acc = b_ref[...]
        for k in range(NBLOCKS):
            acc = acc + _dot(curs[k], w_ref[k * nhid:(k + 1) * nhid, :])
        h = jnp.tanh(acc)

    o = _dot(h, wout_ref[...]) + bout_ref[...]
    nrm = jnp.sqrt(jnp.sum(o * o, axis=1, keepdims=True))
    out_ref[...] = o / jnp.maximum(nrm, 1e-12)


def kernel(x, adj, features, W0, b0, W1, b1, W2, b2, Wout, bout):
    n = adj.shape[0]
    nclass = Wout.shape[1]
    return pl.pallas_call(
        _krylov_body,
        out_shape=jax.ShapeDtypeStruct((n, nclass), jnp.float32),
    )(adj, features, W0, b0.reshape(1, -1), W1, b1.reshape(1, -1),
      W2, b2.reshape(1, -1), Wout, bout.reshape(1, -1))


# speedup vs baseline: 4.1422x; 1.1684x over previous
import jax
import jax.numpy as jnp
from jax.experimental import pallas as pl

NBLOCKS = 4


def _dot(a, b):
    return jax.lax.dot_general(a, b, (((1,), (0,)), ((), ())),
                               preferred_element_type=jnp.float32)


def _apply(A, cur):
    half = A.shape[0] // 2
    top = _dot(A[:half, :], cur)
    bot = _dot(A[half:, :], cur)
    return jnp.concatenate([top, bot], axis=0)


def _krylov_body(adj_ref, feat_ref, w0_ref, b0_ref, w1_ref, b1_ref,
                 w2_ref, b2_ref, wout_ref, bout_ref, out_ref):
    A = adj_ref[...]
    nfeat = feat_ref.shape[1]
    nhid = w0_ref.shape[1]

    curs = [feat_ref[...]]
    for k in range(1, NBLOCKS):
        curs.append(_apply(A, curs[-1]))
    acc = b0_ref[...]
    for k in range(NBLOCKS):
        acc = acc + _dot(curs[k], w0_ref[k * nfeat:(k + 1) * nfeat, :])
    h = jnp.tanh(acc)

    for w_ref, b_ref in ((w1_ref, b1_ref), (w2_ref, b2_ref)):
        curs = [h]
        for k in range(1, NBLOCKS):
            curs.append(_apply(A, curs[-1]))
        acc = b_ref[...]
        for k in range(NBLOCKS):
            acc = acc + _dot(curs[k], w_ref[k * nhid:(k + 1) * nhid, :])
        h = jnp.tanh(acc)

    o = _dot(h, wout_ref[...]) + bout_ref[...]
    nrm = jnp.sqrt(jnp.sum(o * o, axis=1, keepdims=True))
    out_ref[...] = o / jnp.maximum(nrm, 1e-12)


def kernel(x, adj, features, W0, b0, W1, b1, W2, b2, Wout, bout):
    n = adj.shape[0]
    nclass = Wout.shape[1]
    return pl.pallas_call(
        _krylov_body,
        out_shape=jax.ShapeDtypeStruct((n, nclass), jnp.float32),
    )(adj, features, W0, b0.reshape(1, -1), W1, b1.reshape(1, -1),
      W2, b2.reshape(1, -1), Wout, bout.reshape(1, -1))
